# baseline (speedup 1.0000x reference)
import jax
import jax.numpy as jnp
from jax.experimental import pallas as pl

N = 10000
E = 320000
T = 8
D = 128
HID = 128
H = 8
C = 16


def _layernorm(x, g, b, eps=1e-5):
    m = x.mean(-1, keepdims=True)
    v = ((x - m) ** 2).mean(-1, keepdims=True)
    return (x - m) / jnp.sqrt(v + eps) * g + b


def _gat(h, src, dst, W, a_s, a_d, bias):
    xh = (h @ W.T).reshape(N, H, C)
    as_ = (xh * a_s[None]).sum(-1)
    ad_ = (xh * a_d[None]).sum(-1)
    alpha = as_[src] + ad_[dst]
    alpha = jnp.where(alpha > 0, alpha, 0.2 * alpha)
    m = jax.ops.segment_max(alpha, dst, num_segments=N)
    m = jnp.where(jnp.isfinite(m), m, 0.0)
    ex = jnp.exp(alpha - m[dst])
    den = jax.ops.segment_sum(ex, dst, num_segments=N)
    w = ex / (den[dst] + 1e-16)
    out = jax.ops.segment_sum(w[:, :, None] * xh[src], dst, num_segments=N)
    return out.reshape(N, HID) + bias


def _final_kernel(logit_ref, out_ref):
    out_ref[...] = logit_ref[...]


def kernel(x_seq, edge_index, W_proj, b_proj, W_gat, att_src, att_dst, b_gat,
           in_proj_w, in_proj_b, out_proj_w, out_proj_b, ln1_g, ln1_b,
           ln2_g, ln2_b, cW1, cb1, cln_g, cln_b, cW2, cb2):
    loops = jnp.arange(N)
    src = jnp.concatenate([edge_index[0], loops])
    dst = jnp.concatenate([edge_index[1], loops])
    feats = []
    for t in range(T):
        h = jax.nn.relu(x_seq[t] @ W_proj.T + b_proj)
        feats.append(_gat(h, src, dst, W_gat, att_src, att_dst, b_gat))
    nf = jnp.stack(feats, axis=0).transpose(1, 0, 2)
    nx = _layernorm(nf, ln1_g, ln1_b)
    qkv = nx @ in_proj_w.T + in_proj_b
    q, k, v = jnp.split(qkv, 3, axis=-1)
    qh = q.reshape(N, T, H, C).transpose(0, 2, 1, 3)
    kh = k.reshape(N, T, H, C).transpose(0, 2, 1, 3)
    vh = v.reshape(N, T, H, C).transpose(0, 2, 1, 3)
    att = jax.nn.softmax(qh @ kh.transpose(0, 1, 3, 2) / jnp.sqrt(jnp.float32(C)), axis=-1)
    o = (att @ vh).transpose(0, 2, 1, 3).reshape(N, T, HID)
    tf = o @ out_proj_w.T + out_proj_b
    feat = nf + tf
    fused = _layernorm(feat, ln2_g, ln2_b)
    emb = fused[:, -1, :]
    node_ids = (jnp.arange(N, dtype=jnp.float32) / N).reshape(-1, 1)
    emb = emb + jnp.sin(node_ids * 6.28) * 0.1
    noise = jax.random.normal(jax.random.key(1), emb.shape) * 0.05
    emb = emb + noise
    h1 = emb @ cW1.T + cb1
    h1 = _layernorm(h1, cln_g, cln_b)
    h1 = jax.nn.gelu(h1, approximate=False)
    logits = h1 @ cW2.T + cb2
    logits = logits.at[:, 1].add(jnp.sin(jnp.arange(N, dtype=jnp.float32) * 0.5) * 0.2)
    logits = pl.pallas_call(
        _final_kernel,
        out_shape=jax.ShapeDtypeStruct((N, 2), jnp.float32),
    )(logits)
    return (emb, logits)


# speedup vs baseline: 40.9851x; 40.9851x over previous
import functools

import jax
import jax.numpy as jnp
from jax import lax
from jax.experimental import pallas as pl
from jax.experimental.pallas import tpu as pltpu
from jax.experimental.pallas import tpu_sc as plsc

N = 10000
E = 320000
T = 8
D = 128
HID = 128
H = 8
C = 16

NPAD = 10240
DUMMY = 10100
SUB = 16
RPT = NPAD // SUB
K = 48
NCH = 216
SCH = 8
DRPT = 128
EPT = NCH * K
E2P = 32 * EPT

DROW = 2048

BA = 640
BC = 512


def _stage_a_body(x_ref, wp_ref, bp_ref, wg_ref, asw_ref, adw_ref,
                  xh_ref, as_ref, ad_ref):
    x = x_ref[0]
    h = jnp.maximum(
        jnp.dot(x, wp_ref[...], preferred_element_type=jnp.float32)
        + bp_ref[...], 0.0)
    xh = jnp.dot(h, wg_ref[...], preferred_element_type=jnp.float32)
    rows = (pl.program_id(1) * BA
            + lax.broadcasted_iota(jnp.int32, (BA, 1), 0))
    xh = jnp.where(rows < N, xh, 0.0)
    xh_ref[0] = xh
    as_ref[0] = jnp.dot(xh, asw_ref[...], preferred_element_type=jnp.float32)
    ad_ref[0] = jnp.dot(xh, adw_ref[...], preferred_element_type=jnp.float32)


def _stage_a(x_seq, wpT, bp, wgT, asw, adw):
    nblk = NPAD // BA
    return pl.pallas_call(
        _stage_a_body,
        grid=(T, nblk),
        in_specs=[
            pl.BlockSpec((1, BA, D), lambda t, i: (t, i, 0)),
            pl.BlockSpec((D, HID), lambda t, i: (0, 0)),
            pl.BlockSpec((1, HID), lambda t, i: (0, 0)),
            pl.BlockSpec((HID, HID), lambda t, i: (0, 0)),
            pl.BlockSpec((HID, HID), lambda t, i: (0, 0)),
            pl.BlockSpec((HID, HID), lambda t, i: (0, 0)),
        ],
        out_specs=[
            pl.BlockSpec((1, BA, HID), lambda t, i: (t, i, 0)),
            pl.BlockSpec((1, BA, HID), lambda t, i: (t, i, 0)),
            pl.BlockSpec((1, BA, HID), lambda t, i: (t, i, 0)),
        ],
        out_shape=[
            jax.ShapeDtypeStruct((T, NPAD, HID), jnp.float32),
            jax.ShapeDtypeStruct((T, NPAD, HID), jnp.float32),
            jax.ShapeDtypeStruct((T, NPAD, HID), jnp.float32),
        ],
    )(x_seq, wpT, bp, wgT, asw, adw)


def _edge_body(xh_hbm, as_hbm, ad_hbm, src_hbm, dst_hbm, zb_hbm, zs_hbm,
               num_out, den_out,
               sidx_v, didx_v, dstm_v, as_v, ad_v, xh_v, ex_v,
               num_acc, den_acc):
    cid = lax.axis_index("c")
    sid = lax.axis_index("s")
    g = cid * SUB + sid
    row0 = sid * RPT
    drow0 = sid * DRPT

    def t_body(t, carry):
        pltpu.sync_copy(zb_hbm, num_acc.at[pl.ds(row0, RPT)])
        pltpu.sync_copy(zs_hbm, den_acc.at[pl.ds(drow0, DRPT)])
        plsc.subcore_barrier()

        def super_body(jj, c1):
            pltpu.sync_copy(src_hbm.at[g, pl.ds(jj * SCH, SCH)], sidx_v)
            pltpu.sync_copy(dst_hbm.at[g, pl.ds(jj * SCH, SCH)], didx_v)

            def chunk_body(jc, c2):
                sidx = sidx_v.at[jc]
                didx = didx_v.at[jc]
                pltpu.sync_copy(as_hbm.at[t].at[sidx], as_v)
                pltpu.sync_copy(ad_hbm.at[t].at[didx], ad_v)
                pltpu.sync_copy(xh_hbm.at[t].at[sidx], xh_v)

                def grp_body(gg, c3):
                    base = gg * 16
                    dstv = didx_v[jc, pl.ds(base, 16)]
                    dstm_v[0, pl.ds(base, 16)] = lax.bitwise_and(dstv, 2047)
                    for l in range(16):
                        e = base + l
                        a = as_v[e, pl.ds(0, 16)]
                        d = ad_v[e, pl.ds(0, 16)]
                        al = a + d
                        ex = jnp.exp(jnp.maximum(al, 0.2 * al))
                        sel = lax.shift_right_logical(dstv[l], 11)
                        for q in range(8):
                            ex_v[e, pl.ds(q * 16, 16)] = jnp.where(
                                sel == q, ex, jnp.zeros((16,), jnp.float32))
                            s2 = ex[q]
                            row = xh_v[e, pl.ds(q * 16, 16)]
                            xh_v[e, pl.ds(q * 16, 16)] = row * s2
                    return c3

                lax.fori_loop(0, K // 16, grp_body, 0)
                pltpu.sync_copy(ex_v, den_acc.at[dstm_v.at[0]], add=True)
                pltpu.sync_copy(xh_v, num_acc.at[didx], add=True)
                return c2

            lax.fori_loop(0, SCH, chunk_body, 0)
            return c1

        lax.fori_loop(0, NCH // SCH, super_body, 0)
        plsc.subcore_barrier()
        pltpu.sync_copy(num_acc.at[pl.ds(row0, RPT)],
                        num_out.at[cid, t, pl.ds(row0, RPT)])
        pltpu.sync_copy(den_acc.at[pl.ds(drow0, DRPT)],
                        den_out.at[cid, t, pl.ds(drow0, DRPT)])
        return carry

    lax.fori_loop(0, T, t_body, 0)


def _edge_aggregate(xh, as16, ad16, srcK, dstK):
    zb = jnp.zeros((RPT, HID), jnp.float32)
    zs = jnp.zeros((DRPT, HID), jnp.float32)
    mesh = plsc.VectorSubcoreMesh(core_axis_name="c", subcore_axis_name="s")
    fn = functools.partial(
        pl.kernel, mesh=mesh,
        out_type=(
            jax.ShapeDtypeStruct((2, T, NPAD, HID), jnp.float32),
            jax.ShapeDtypeStruct((2, T, DROW, HID), jnp.float32),
        ),
        scratch_types=[
            pltpu.VMEM((SCH, K), jnp.int32),
            pltpu.VMEM((SCH, K), jnp.int32),
            pltpu.VMEM((1, K), jnp.int32),
            pltpu.VMEM((K, HID), jnp.float32),
            pltpu.VMEM((K, HID), jnp.float32),
            pltpu.VMEM((K, HID), jnp.float32),
            pltpu.VMEM((K, HID), jnp.float32),
            pltpu.VMEM_SHARED((NPAD, HID), jnp.float32),
            pltpu.VMEM_SHARED((DROW, HID), jnp.float32),
        ],
    )(_edge_body)
    return fn(xh, as16, ad16, srcK, dstK, zb, zs)


def _stage_c_body(num_ref, den_ref, mq_ref, bgat_ref, g1_ref, b1_ref,
                  winT_ref, bin_ref, woutT_ref, bout_ref, g2_ref, b2_ref,
                  ee_ref, w1T_ref, cb1_ref, gc_ref, bcl_ref, w2T_ref, le_ref,
                  r16_ref, es16_ref, emb_ref, log_ref):
    r16 = r16_ref[...]
    nsum = num_ref[0] + num_ref[1]
    dsum = den_ref[0] + den_ref[1]
    n2 = nsum.reshape(T * BC, HID)
    d2 = dsum.reshape(T * BC, HID)
    db = jnp.dot(d2, mq_ref[0], preferred_element_type=jnp.float32)
    nf = n2 / (db + 1e-16) + bgat_ref[...]
    mu = nf.mean(-1, keepdims=True)
    var = ((nf - mu) ** 2).mean(-1, keepdims=True)
    nx = (nf - mu) * lax.rsqrt(var + 1e-5) * g1_ref[...] + b1_ref[...]
    qkv = jnp.dot(nx, winT_ref[...], preferred_element_type=jnp.float32) \
        + bin_ref[...]
    qkv3 = qkv.reshape(T, BC, 3 * HID)
    q7 = qkv3[T - 1, :, 0:HID]
    es = es16_ref[...]
    sj = [jnp.dot(q7 * qkv3[j, :, HID:2 * HID], es,
                  preferred_element_type=jnp.float32) * 0.25
          for j in range(T)]
    mx = sj[0]
    for j in range(1, T):
        mx = jnp.maximum(mx, sj[j])
    ejs = [jnp.exp(s - mx) for s in sj]
    dena = ejs[0]
    for j in range(1, T):
        dena = dena + ejs[j]
    rden = 1.0 / dena
    o7 = jnp.zeros((BC, HID), jnp.float32)
    for j in range(T):
        p = jnp.dot(ejs[j] * rden, r16, preferred_element_type=jnp.float32)
        o7 = o7 + p * qkv3[j, :, 2 * HID:3 * HID]
    tf7 = jnp.dot(o7, woutT_ref[...], preferred_element_type=jnp.float32) \
        + bout_ref[...]
    nf7 = nf.reshape(T, BC, HID)[T - 1]
    feat = nf7 + tf7
    mu2 = feat.mean(-1, keepdims=True)
    v2 = ((feat - mu2) ** 2).mean(-1, keepdims=True)
    fused = (feat - mu2) * lax.rsqrt(v2 + 1e-5) * g2_ref[...] + b2_ref[...]
    emb = fused + ee_ref[...]
    emb_ref[...] = emb
    h1 = jnp.dot(emb, w1T_ref[...], preferred_element_type=jnp.float32) \
        + cb1_ref[...]
    mu3 = h1.mean(-1, keepdims=True)
    v3 = ((h1 - mu3) ** 2).mean(-1, keepdims=True)
    h1 = (h1 - mu3) * lax.rsqrt(v3 + 1e-5) * gc_ref[...] + bcl_ref[...]
    h1 = 0.5 * h1 * (1.0 + lax.erf(h1 * (2.0 ** -0.5)))
    lg = jnp.dot(h1, w2T_ref[...], preferred_element_type=jnp.float32)
    log_ref[...] = lg[:, 0:2] + le_ref[...]


def _stage_c(num, den, mq, b_gat, g1, b1, winT, bin_, woutT, bout, g2, b2,
             ee, w1T, cb1, gc, bcl, w2T8, le, r16, es16):
    nblk = NPAD // BC
    full = lambda shape: pl.BlockSpec(shape, lambda i: tuple(0 for _ in shape))
    return pl.pallas_call(
        _stage_c_body,
        grid=(nblk,),
        in_specs=[
            pl.BlockSpec((2, T, BC, HID), lambda i: (0, 0, i, 0)),
            pl.BlockSpec((2, T, BC, HID), lambda i: (0, 0, i % 4, 0)),
            pl.BlockSpec((1, HID, HID), lambda i: (i // 4, 0, 0)),
            full((1, HID)), full((1, HID)), full((1, HID)),
            full((HID, 3 * HID)), full((1, 3 * HID)),
            full((HID, HID)), full((1, HID)),
            full((1, HID)), full((1, HID)),
            pl.BlockSpec((BC, HID), lambda i: (i, 0)),
            full((HID, HID)), full((1, HID)),
            full((1, HID)), full((1, HID)),
            full((HID, 8)),
            pl.BlockSpec((BC, 2), lambda i: (i, 0)),
            full((16, HID)), full((HID, 16)),
        ],
        out_specs=[
            pl.BlockSpec((BC, HID), lambda i: (i, 0)),
            pl.BlockSpec((BC, 2), lambda i: (i, 0)),
        ],
        out_shape=[
            jax.ShapeDtypeStruct((N, HID), jnp.float32),
            jax.ShapeDtypeStruct((N, 2), jnp.float32),
        ],
    )(num, den, mq, b_gat, g1, b1, winT, bin_, woutT, bout, g2, b2, ee,
      w1T, cb1, gc, bcl, w2T8, le, r16, es16)


def kernel(x_seq, edge_index, W_proj, b_proj, W_gat, att_src, att_dst, b_gat,
           in_proj_w, in_proj_b, out_proj_w, out_proj_b, ln1_g, ln1_b,
           ln2_g, ln2_b, cW1, cb1, cln_g, cln_b, cW2, cb2):
    ii = jnp.arange(HID, dtype=jnp.int32)
    asw = jnp.tile(jnp.zeros((HID, 16), jnp.float32).at[ii, ii // 16].set(
        att_src.reshape(-1)), (1, 8))
    adw = jnp.tile(jnp.zeros((HID, 16), jnp.float32).at[ii, ii // 16].set(
        att_dst.reshape(-1)), (1, 8))
    r16 = jnp.zeros((16, HID), jnp.float32).at[ii // 16, ii].set(1.0)
    es16 = jnp.zeros((HID, 16), jnp.float32).at[ii, ii // 16].set(1.0)

    xh, as16, ad16 = _stage_a(x_seq, W_proj.T, b_proj.reshape(1, HID),
                              W_gat.T, asw, adw)

    pad = E2P - E - N
    loops = jnp.arange(N, dtype=jnp.int32)
    fill = jnp.full((pad,), DUMMY, jnp.int32)
    srcK = jnp.concatenate([edge_index[0].astype(jnp.int32), loops, fill]
                           ).reshape(32, NCH, K)
    dstK = jnp.concatenate([edge_index[1].astype(jnp.int32), loops, fill]
                           ).reshape(32, NCH, K)

    num, den = _edge_aggregate(xh, as16, ad16, srcK, dstK)

    node_ids = (jnp.arange(N, dtype=jnp.float32) / N).reshape(-1, 1)
    noise = jax.random.normal(jax.random.key(1), (N, HID)) * 0.05
    ee = jnp.sin(node_ids * 6.28) * 0.1 + noise
    le = cb2.reshape(1, 2) + jnp.stack(
        [jnp.zeros((N,), jnp.float32),
         jnp.sin(jnp.arange(N, dtype=jnp.float32) * 0.5) * 0.2], axis=1)
    w2T8 = jnp.zeros((HID, 8), jnp.float32).at[:, 0:2].set(cW2.T)

    qs = jnp.repeat(jnp.arange(8, dtype=jnp.int32), 128)
    hs = jnp.tile(jnp.repeat(jnp.arange(8, dtype=jnp.int32), 16), 8)
    cs = jnp.tile(jnp.arange(16, dtype=jnp.int32), 64)
    mq = jnp.zeros((8, HID, HID), jnp.float32).at[
        qs, qs * 16 + hs, hs * 16 + cs].set(1.0)

    emb, logits = _stage_c(
        num, den, mq, b_gat.reshape(1, HID), ln1_g.reshape(1, HID),
        ln1_b.reshape(1, HID), in_proj_w.T, in_proj_b.reshape(1, 3 * HID),
        out_proj_w.T, out_proj_b.reshape(1, HID), ln2_g.reshape(1, HID),
        ln2_b.reshape(1, HID), ee, cW1.T, cb1.reshape(1, HID),
        cln_g.reshape(1, HID), cln_b.reshape(1, HID), w2T8, le, r16, es16)
    return (emb, logits)


# speedup vs baseline: 57.4566x; 1.4019x over previous
import functools

import jax
import jax.numpy as jnp
from jax import lax
from jax.experimental import pallas as pl
from jax.experimental.pallas import tpu as pltpu
from jax.experimental.pallas import tpu_sc as plsc

N = 10000
E = 320000
T = 8
D = 128
HID = 128
H = 8
C = 16

NPAD = 10240
DUMMY = 10100
SUB = 16
RPT = NPAD // SUB
K = 16
NCH = 648
GS = 24
NSUP = 27
DRPT = 80
EPT = NCH * K
E2P = 32 * EPT

DROW = 1280

BA = 640
BC = 256


def _stage_a_body(x_ref, wp_ref, bp_ref, wg_ref, asw_ref, adw_ref,
                  xh_ref, as_ref, ad_ref):
    x = x_ref[0]
    h = jnp.maximum(
        jnp.dot(x, wp_ref[...], preferred_element_type=jnp.float32, precision=lax.Precision.HIGHEST)
        + bp_ref[...], 0.0)
    xh = jnp.dot(h, wg_ref[...], preferred_element_type=jnp.float32, precision=lax.Precision.HIGHEST)
    rows = (pl.program_id(1) * BA
            + lax.broadcasted_iota(jnp.int32, (BA, 1), 0))
    xh = jnp.where(rows < N, xh, 0.0)
    xh_ref[0] = xh
    as_ref[0] = jnp.dot(xh, asw_ref[...], preferred_element_type=jnp.float32, precision=lax.Precision.HIGHEST)
    ad_ref[0] = jnp.dot(xh, adw_ref[...], preferred_element_type=jnp.float32, precision=lax.Precision.HIGHEST)


def _stage_a(x_seq, wpT, bp, wgT, asw, adw):
    nblk = NPAD // BA
    return pl.pallas_call(
        _stage_a_body,
        grid=(T, nblk),
        in_specs=[
            pl.BlockSpec((1, BA, D), lambda t, i: (t, i, 0)),
            pl.BlockSpec((D, HID), lambda t, i: (0, 0)),
            pl.BlockSpec((1, HID), lambda t, i: (0, 0)),
            pl.BlockSpec((HID, HID), lambda t, i: (0, 0)),
            pl.BlockSpec((HID, HID), lambda t, i: (0, 0)),
            pl.BlockSpec((HID, HID), lambda t, i: (0, 0)),
        ],
        out_specs=[
            pl.BlockSpec((1, BA, HID), lambda t, i: (t, i, 0)),
            pl.BlockSpec((1, BA, HID), lambda t, i: (t, i, 0)),
            pl.BlockSpec((1, BA, HID), lambda t, i: (t, i, 0)),
        ],
        out_shape=[
            jax.ShapeDtypeStruct((T, NPAD, HID), jnp.float32),
            jax.ShapeDtypeStruct((T, NPAD, HID), jnp.float32),
            jax.ShapeDtypeStruct((T, NPAD, HID), jnp.float32),
        ],
    )(x_seq, wpT, bp, wgT, asw, adw)


def _edge_body(xh_hbm, as_hbm, ad_hbm, src_hbm, dst_hbm, zb_hbm, zs_hbm,
               num_out, den_out,
               sidx_v, didx_v, dstm_v,
               as_g0, as_g1, ad_g0, ad_g1, xh_g0, xh_g1,
               xs_s0, xs_s1, ex_s0, ex_s1,
               sem_g0, sem_g1, sem_s0, sem_s1,
               num_acc, den_acc):
    cid = lax.axis_index("c")
    sid = lax.axis_index("s")
    g = cid * SUB + sid
    row0 = sid * RPT
    drow0 = sid * DRPT
    as_g = (as_g0, as_g1)
    ad_g = (ad_g0, ad_g1)
    xh_g = (xh_g0, xh_g1)
    xs_s = (xs_s0, xs_s1)
    ex_s = (ex_s0, ex_s1)
    sem_g = (sem_g0, sem_g1)
    sem_s = (sem_s0, sem_s1)

    def start_gather(t, r, s):
        sidx = sidx_v.at[r]
        didx = didx_v.at[r]
        pltpu.async_copy(as_hbm.at[t].at[sidx], as_g[s], sem_g[s])
        pltpu.async_copy(ad_hbm.at[t].at[didx], ad_g[s], sem_g[s])
        pltpu.async_copy(xh_hbm.at[t].at[sidx], xh_g[s], sem_g[s])

    def wait_gather(s):
        pltpu.make_async_copy(as_hbm.at[0].at[sidx_v.at[0]],
                              as_g[s], sem_g[s]).wait()
        pltpu.make_async_copy(ad_hbm.at[0].at[didx_v.at[0]],
                              ad_g[s], sem_g[s]).wait()
        pltpu.make_async_copy(xh_hbm.at[0].at[sidx_v.at[0]],
                              xh_g[s], sem_g[s]).wait()

    def start_scatter(r, s):
        pltpu.async_copy(ex_s[s], den_acc.at[dstm_v.at[s]], sem_s[s],
                         add=True)
        pltpu.async_copy(xs_s[s], num_acc.at[didx_v.at[r]], sem_s[s],
                         add=True)

    def wait_scatter(s):
        pltpu.make_async_copy(ex_s[s], den_acc.at[dstm_v.at[s]],
                              sem_s[s]).wait()
        pltpu.make_async_copy(xs_s[s], num_acc.at[didx_v.at[0]],
                              sem_s[s]).wait()

    def compute(r, s):
        dstv = didx_v[r, pl.ds(0, 16)]
        qv = lax.shift_right_logical(
            lax.shift_right_logical(dstv, 8) * 205, 10)
        dstm_v[s, pl.ds(0, 16)] = dstv - qv * 1280
        for l in range(K):
            a = as_g[s][l, pl.ds(0, 16)]
            d = ad_g[s][l, pl.ds(0, 16)]
            al = a + d
            ex = jnp.exp(jnp.maximum(al, 0.2 * al))
            sel = qv[l]
            for q in range(8):
                ex_s[s][l, pl.ds(q * 16, 16)] = jnp.where(
                    sel == q, ex, jnp.zeros((16,), jnp.float32))
                xs_s[s][l, pl.ds(q * 16, 16)] = \
                    xh_g[s][l, pl.ds(q * 16, 16)] * ex[q]

    def t_body(t, carry):
        pltpu.sync_copy(zb_hbm, num_acc.at[pl.ds(row0, RPT)])
        pltpu.sync_copy(zs_hbm, den_acc.at[pl.ds(drow0, DRPT)])
        plsc.subcore_barrier()

        def super_body(jj, c1):
            @pl.when(jj > 0)
            def _():
                wait_scatter(0)
                wait_scatter(1)
            pltpu.sync_copy(src_hbm.at[g, pl.ds(jj * GS, GS)], sidx_v)
            pltpu.sync_copy(dst_hbm.at[g, pl.ds(jj * GS, GS)], didx_v)
            start_gather(t, 0, 0)

            def pair_body(p, c2):
                for b in (0, 1):
                    jc = p * 2 + b
                    wait_gather(b)
                    if b == 0:
                        start_gather(t, jc + 1, 1)
                    else:
                        @pl.when(p < GS // 2 - 1)
                        def _():
                            start_gather(t, jc + 1, 0)
                    @pl.when(p > 0)
                    def _():
                        wait_scatter(b)
                    compute(jc, b)
                    start_scatter(jc, b)
                return c2

            lax.fori_loop(0, GS // 2, pair_body, 0)
            return c1

        lax.fori_loop(0, NSUP, super_body, 0)
        wait_scatter(0)
        wait_scatter(1)
        plsc.subcore_barrier()
        pltpu.sync_copy(num_acc.at[pl.ds(row0, RPT)],
                        num_out.at[cid, t, pl.ds(row0, RPT)])
        pltpu.sync_copy(den_acc.at[pl.ds(drow0, DRPT)],
                        den_out.at[cid, t, pl.ds(drow0, DRPT)])
        return carry

    lax.fori_loop(0, T, t_body, 0)


def _edge_aggregate(xh, as16, ad16, srcK, dstK):
    zb = jnp.zeros((RPT, HID), jnp.float32)
    zs = jnp.zeros((DRPT, HID), jnp.float32)
    mesh = plsc.VectorSubcoreMesh(core_axis_name="c", subcore_axis_name="s")
    buf = lambda: pltpu.VMEM((K, HID), jnp.float32)
    fn = functools.partial(
        pl.kernel, mesh=mesh,
        out_type=(
            jax.ShapeDtypeStruct((2, T, NPAD, HID), jnp.float32),
            jax.ShapeDtypeStruct((2, T, DROW, HID), jnp.float32),
        ),
        scratch_types=[
            pltpu.VMEM((GS, K), jnp.int32),
            pltpu.VMEM((GS, K), jnp.int32),
            pltpu.VMEM((2, K), jnp.int32),
            buf(), buf(), buf(), buf(), buf(), buf(),
            buf(), buf(), buf(), buf(),
            pltpu.SemaphoreType.DMA,
            pltpu.SemaphoreType.DMA,
            pltpu.SemaphoreType.DMA,
            pltpu.SemaphoreType.DMA,
            pltpu.VMEM_SHARED((NPAD, HID), jnp.float32),
            pltpu.VMEM_SHARED((DROW, HID), jnp.float32),
        ],
    )(_edge_body)
    return fn(xh, as16, ad16, srcK, dstK, zb, zs)


def _stage_c_body(num_ref, den_ref, mq_ref, bgat_ref, g1_ref, b1_ref,
                  winT_ref, bin_ref, woutT_ref, bout_ref, g2_ref, b2_ref,
                  ee_ref, w1T_ref, cb1_ref, gc_ref, bcl_ref, w2T_ref, le_ref,
                  r16_ref, es16_ref, emb_ref, log_ref):
    r16 = r16_ref[...]
    nsum = num_ref[0] + num_ref[1]
    dsum = den_ref[0] + den_ref[1]
    n2 = nsum.reshape(T * BC, HID)
    d2 = dsum.reshape(T * BC, HID)
    db = jnp.dot(d2, mq_ref[0], preferred_element_type=jnp.float32, precision=lax.Precision.HIGHEST)
    nf = n2 / (db + 1e-16) + bgat_ref[...]
    mu = nf.mean(-1, keepdims=True)
    var = ((nf - mu) ** 2).mean(-1, keepdims=True)
    nx = (nf - mu) * lax.rsqrt(var + 1e-5) * g1_ref[...] + b1_ref[...]
    qkv = jnp.dot(nx, winT_ref[...], preferred_element_type=jnp.float32, precision=lax.Precision.HIGHEST) \
        + bin_ref[...]
    qkv3 = qkv.reshape(T, BC, 3 * HID)
    q7 = qkv3[T - 1, :, 0:HID]
    es = es16_ref[...]
    sj = [jnp.dot(q7 * qkv3[j, :, HID:2 * HID], es,
                  preferred_element_type=jnp.float32, precision=lax.Precision.HIGHEST) * 0.25
          for j in range(T)]
    mx = sj[0]
    for j in range(1, T):
        mx = jnp.maximum(mx, sj[j])
    ejs = [jnp.exp(s - mx) for s in sj]
    dena = ejs[0]
    for j in range(1, T):
        dena = dena + ejs[j]
    rden = 1.0 / dena
    o7 = jnp.zeros((BC, HID), jnp.float32)
    for j in range(T):
        p = jnp.dot(ejs[j] * rden, r16, preferred_element_type=jnp.float32, precision=lax.Precision.HIGHEST)
        o7 = o7 + p * qkv3[j, :, 2 * HID:3 * HID]
    tf7 = jnp.dot(o7, woutT_ref[...], preferred_element_type=jnp.float32, precision=lax.Precision.HIGHEST) \
        + bout_ref[...]
    nf7 = nf.reshape(T, BC, HID)[T - 1]
    feat = nf7 + tf7
    mu2 = feat.mean(-1, keepdims=True)
    v2 = ((feat - mu2) ** 2).mean(-1, keepdims=True)
    fused = (feat - mu2) * lax.rsqrt(v2 + 1e-5) * g2_ref[...] + b2_ref[...]
    emb = fused + ee_ref[...]
    emb_ref[...] = emb
    h1 = jnp.dot(emb, w1T_ref[...], preferred_element_type=jnp.float32, precision=lax.Precision.HIGHEST) \
        + cb1_ref[...]
    mu3 = h1.mean(-1, keepdims=True)
    v3 = ((h1 - mu3) ** 2).mean(-1, keepdims=True)
    h1 = (h1 - mu3) * lax.rsqrt(v3 + 1e-5) * gc_ref[...] + bcl_ref[...]
    h1 = 0.5 * h1 * (1.0 + lax.erf(h1 * (2.0 ** -0.5)))
    lg = jnp.dot(h1, w2T_ref[...], preferred_element_type=jnp.float32, precision=lax.Precision.HIGHEST)
    log_ref[...] = lg[:, 0:2] + le_ref[...]


def _stage_c(num, den, mq, b_gat, g1, b1, winT, bin_, woutT, bout, g2, b2,
             ee, w1T, cb1, gc, bcl, w2T8, le, r16, es16):
    nblk = NPAD // BC
    full = lambda shape: pl.BlockSpec(shape, lambda i: tuple(0 for _ in shape))
    return pl.pallas_call(
        _stage_c_body,
        grid=(nblk,),
        in_specs=[
            pl.BlockSpec((2, T, BC, HID), lambda i: (0, 0, i, 0)),
            pl.BlockSpec((2, T, BC, HID), lambda i: (0, 0, i % 5, 0)),
            pl.BlockSpec((1, HID, HID), lambda i: (i // 5, 0, 0)),
            full((1, HID)), full((1, HID)), full((1, HID)),
            full((HID, 3 * HID)), full((1, 3 * HID)),
            full((HID, HID)), full((1, HID)),
            full((1, HID)), full((1, HID)),
            pl.BlockSpec((BC, HID), lambda i: (i, 0)),
            full((HID, HID)), full((1, HID)),
            full((1, HID)), full((1, HID)),
            full((HID, 8)),
            pl.BlockSpec((BC, 2), lambda i: (i, 0)),
            full((16, HID)), full((HID, 16)),
        ],
        out_specs=[
            pl.BlockSpec((BC, HID), lambda i: (i, 0)),
            pl.BlockSpec((BC, 2), lambda i: (i, 0)),
        ],
        out_shape=[
            jax.ShapeDtypeStruct((N, HID), jnp.float32),
            jax.ShapeDtypeStruct((N, 2), jnp.float32),
        ],
    )(num, den, mq, b_gat, g1, b1, winT, bin_, woutT, bout, g2, b2, ee,
      w1T, cb1, gc, bcl, w2T8, le, r16, es16)


def kernel(x_seq, edge_index, W_proj, b_proj, W_gat, att_src, att_dst, b_gat,
           in_proj_w, in_proj_b, out_proj_w, out_proj_b, ln1_g, ln1_b,
           ln2_g, ln2_b, cW1, cb1, cln_g, cln_b, cW2, cb2):
    ii = jnp.arange(HID, dtype=jnp.int32)
    asw = jnp.tile(jnp.zeros((HID, 16), jnp.float32).at[ii, ii // 16].set(
        att_src.reshape(-1)), (1, 8))
    adw = jnp.tile(jnp.zeros((HID, 16), jnp.float32).at[ii, ii // 16].set(
        att_dst.reshape(-1)), (1, 8))
    r16 = jnp.zeros((16, HID), jnp.float32).at[ii // 16, ii].set(1.0)
    es16 = jnp.zeros((HID, 16), jnp.float32).at[ii, ii // 16].set(1.0)

    xh, as16, ad16 = _stage_a(x_seq, W_proj.T, b_proj.reshape(1, HID),
                              W_gat.T, asw, adw)

    pad = E2P - E - N
    loops = jnp.arange(N, dtype=jnp.int32)
    fill = jnp.full((pad,), DUMMY, jnp.int32)
    srcK = jnp.concatenate([edge_index[0].astype(jnp.int32), loops, fill]
                           ).reshape(32, NCH, K)
    dstK = jnp.concatenate([edge_index[1].astype(jnp.int32), loops, fill]
                           ).reshape(32, NCH, K)

    num, den = _edge_aggregate(xh, as16, ad16, srcK, dstK)

    node_ids = (jnp.arange(N, dtype=jnp.float32) / N).reshape(-1, 1)
    noise = jax.random.normal(jax.random.key(1), (N, HID)) * 0.05
    ee = jnp.sin(node_ids * 6.28) * 0.1 + noise
    le = cb2.reshape(1, 2) + jnp.stack(
        [jnp.zeros((N,), jnp.float32),
         jnp.sin(jnp.arange(N, dtype=jnp.float32) * 0.5) * 0.2], axis=1)
    w2T8 = jnp.zeros((HID, 8), jnp.float32).at[:, 0:2].set(cW2.T)

    qs = jnp.repeat(jnp.arange(8, dtype=jnp.int32), 128)
    hs = jnp.tile(jnp.repeat(jnp.arange(8, dtype=jnp.int32), 16), 8)
    cs = jnp.tile(jnp.arange(16, dtype=jnp.int32), 64)
    mq = jnp.zeros((8, HID, HID), jnp.float32).at[
        qs, qs * 16 + hs, hs * 16 + cs].set(1.0)

    emb, logits = _stage_c(
        num, den, mq, b_gat.reshape(1, HID), ln1_g.reshape(1, HID),
        ln1_b.reshape(1, HID), in_proj_w.T, in_proj_b.reshape(1, 3 * HID),
        out_proj_w.T, out_proj_b.reshape(1, HID), ln2_g.reshape(1, HID),
        ln2_b.reshape(1, HID), ee, cW1.T, cb1.reshape(1, HID),
        cln_g.reshape(1, HID), cln_b.reshape(1, HID), w2T8, le, r16, es16)
    return (emb, logits)
